# baseline (speedup 1.0000x reference)
import jax
import jax.numpy as jnp
from jax import lax
from jax.experimental import pallas as pl
from jax.experimental.pallas import tpu as pltpu
from jax.experimental.pallas import tpu_sc as plsc

B, D, H, W = 4, 16, 512, 512
K = 32
N = H * W
NCORE, NS, L = 2, 16, 16
BPC = B // NCORE
SPAN = N // NS
CHUNK = 2048
NCK = SPAN // CHUNK
NPAIR = NCK // 2
GRP = CHUNK // L

DELTA_VAR = 0.5
DELTA_DIST = 1.5
REG_W = 0.001

_mesh = plsc.VectorSubcoreMesh(core_axis_name="c", subcore_axis_name="s")


def _sqrt16(s2):
    s2 = jnp.maximum(s2, 1e-30)
    y = plsc.bitcast(0x5F3759DF - (plsc.bitcast(s2, jnp.int32) >> 1),
                     jnp.float32)
    for _ in range(2):
        y = y * (1.5 - 0.5 * s2 * y * y)
    return s2 * y


def _sc_all(emb_hbm, gt_hbm, out_hbm, st_sums, st_cnt, st_pacc, ebuf,
            gbuf, sums, cnt, cnt2, cbuf, pacc, tmp, tmpc, ovec, sems):
    cid = lax.axis_index("c")
    sid = lax.axis_index("s")
    zeros = jnp.zeros((L,), jnp.float32)
    ones = jnp.ones((L,), jnp.float32)
    lane = lax.iota(jnp.int32, L)

    def issue(bb, ck, buf):
        bg = cid * BPC + bb
        off = sid * SPAN + ck * CHUNK
        de = pltpu.async_copy(emb_hbm.at[bg, :, pl.ds(off, CHUNK)],
                              ebuf.at[buf], sems.at[buf])
        dg = pltpu.async_copy(gt_hbm.at[bg, pl.ds(off, CHUNK)],
                              gbuf.at[buf], sems.at[buf])
        return de, dg

    def wait(bb, ck, buf):
        bg = cid * BPC + bb
        off = sid * SPAN + ck * CHUNK
        pltpu.make_async_copy(emb_hbm.at[bg, :, pl.ds(off, CHUNK)],
                              ebuf.at[buf], sems.at[buf]).wait()
        pltpu.make_async_copy(gt_hbm.at[bg, pl.ds(off, CHUNK)],
                              gbuf.at[buf], sems.at[buf]).wait()

    def stream(bb, compute):
        issue(bb, 0, 0)
        issue(bb, 1, 1)

        def pair(p, _):
            wait(bb, 2 * p, 0)
            compute(0)
            issue(bb, 2 * p + 2, 0)
            wait(bb, 2 * p + 1, 1)
            compute(1)
            issue(bb, 2 * p + 3, 1)
            return 0

        lax.fori_loop(0, NPAIR - 1, pair, 0)
        wait(bb, NCK - 2, 0)
        compute(0)
        wait(bb, NCK - 1, 1)
        compute(1)

    for bb in range(BPC):
        for j in range(K * D // L):
            sums[pl.ds(j * L, L)] = zeros
        for j in range(K // L):
            cnt[pl.ds(j * L, L)] = zeros

        def compute1(buf):
            @plsc.parallel_loop(0, GRP, unroll=2)
            def _(i):
                o = i * L
                seg = gbuf[buf, pl.ds(o, L)]
                segd = seg << 4
                vs = [ebuf[buf, d, pl.ds(o, L)] for d in range(D)]
                plsc.addupdate_scatter(cnt, [seg], ones)
                for d in range(D):
                    plsc.addupdate_scatter(sums, [segd + d], vs[d])

        stream(bb, compute1)
        pltpu.sync_copy(sums, st_sums.at[cid, bb, sid])
        pltpu.sync_copy(cnt, st_cnt.at[cid, bb, sid])

    plsc.subcore_barrier()

    for bb in range(BPC):
        pltpu.sync_copy(st_sums.at[cid, bb], tmp)
        pltpu.sync_copy(st_cnt.at[cid, bb], tmpc)

        def red_sums(j, _):
            acc = tmp[0, pl.ds(j * L, L)]
            for w in range(1, NS):
                acc = acc + tmp[w, pl.ds(j * L, L)]
            sums[pl.ds(j * L, L)] = acc
            return 0

        lax.fori_loop(0, K * D // L, red_sums, 0)
        for j in range(K // L):
            acc = tmpc[0, pl.ds(j * L, L)]
            for w in range(1, NS):
                acc = acc + tmpc[w, pl.ds(j * L, L)]
            cnt2[bb, pl.ds(j * L, L)] = acc

        def cen(k, _, bb=bb):
            ck = plsc.load_gather(cnt2, [jnp.full((L,), bb, jnp.int32),
                                         jnp.full((L,), k, jnp.int32)])
            safe = jnp.where(ck > 0.0, ck, 1.0)
            cbuf[pl.ds(bb * K * D + k * D, L)] = (
                sums[pl.ds(k * D, L)] / safe)
            return 0

        lax.fori_loop(0, K, cen, 0)

    for bb in range(BPC):
        for j in range(K // L):
            pacc[pl.ds(j * L, L)] = zeros

        def compute2(buf, bb=bb):
            @plsc.parallel_loop(0, GRP, unroll=4)
            def _(i):
                o = i * L
                seg = gbuf[buf, pl.ds(o, L)]
                segd = seg << 4
                vs = [ebuf[buf, d, pl.ds(o, L)] for d in range(D)]
                cs = [plsc.load_gather(cbuf, [segd + (bb * K * D + d)])
                      for d in range(D)]
                ps = []
                for j in range(4):
                    t0 = vs[j] - cs[j]
                    p = t0 * t0
                    for d in range(j + 4, D, 4):
                        td = vs[d] - cs[d]
                        p = p + td * td
                    ps.append(p)
                s2 = (ps[0] + ps[1]) + (ps[2] + ps[3])
                r = jnp.maximum(_sqrt16(s2) - DELTA_VAR, 0.0)
                plsc.addupdate_scatter(pacc, [seg], r * r)

        stream(bb, compute2)
        pltpu.sync_copy(pacc, st_pacc.at[cid, bb, sid])

    plsc.subcore_barrier()

    @pl.when(sid == 0)
    def _final():
        loss = zeros
        for bb in range(BPC):
            pltpu.sync_copy(st_pacc.at[cid, bb], tmpc)
            kf = zeros
            pullnum = zeros
            pres = []
            for j in range(K // L):
                pinst = tmpc[0, pl.ds(j * L, L)]
                for w in range(1, NS):
                    pinst = pinst + tmpc[w, pl.ds(j * L, L)]
                ck = cnt2[bb, pl.ds(j * L, L)]
                p = (ck > 0.0).astype(jnp.float32)
                pres.append(p)
                safe = jnp.where(ck > 0.0, ck, 1.0)
                kf = kf + p
                pullnum = pullnum + pinst / safe
            kf_safe = jnp.maximum(jnp.full((L,), jnp.sum(kf)), 1.0)
            pull = jnp.full((L,), jnp.sum(pullnum)) / kf_safe

            cj = [[plsc.load_gather(
                cbuf, [(jb * L + lane) * D + (bb * K * D + d)])
                for jb in range(K // L)] for d in range(D)]
            push_sum = zeros
            npair = zeros
            regsum = zeros
            for jb in range(K // L):
                n2 = cj[0][jb] * cj[0][jb]
                for d in range(1, D):
                    n2 = n2 + cj[d][jb] * cj[d][jb]
                regsum = regsum + pres[jb] * _sqrt16(n2)

            def pair_body(i, carry, bb=bb, pres=pres, cj=cj):
                push_sum, npair = carry
                sp = [plsc.load_gather(
                    cbuf, [jnp.full((L,), bb * K * D, jnp.int32)
                           + i * D + d]) for d in range(D)]
                pi = (plsc.load_gather(
                    cnt2, [jnp.full((L,), bb, jnp.int32),
                           jnp.full((L,), 0, jnp.int32) + i])
                    > 0.0).astype(jnp.float32)
                for jb in range(K // L):
                    t0 = sp[0] - cj[0][jb]
                    d2 = t0 * t0
                    for d in range(1, D):
                        td = sp[d] - cj[d][jb]
                        d2 = d2 + td * td
                    dist = _sqrt16(d2)
                    notself = (lane + (jb * L) != i).astype(jnp.float32)
                    pm = pres[jb] * pi * notself
                    rr = jnp.maximum(2.0 * DELTA_DIST - dist, 0.0)
                    push_sum = push_sum + pm * (rr * rr)
                    npair = npair + pm
                return push_sum, npair

            push_sum, npair = lax.fori_loop(0, K, pair_body,
                                            (push_sum, npair))
            nps = jnp.full((L,), jnp.sum(npair))
            push = jnp.where(
                nps > 0.0,
                jnp.full((L,), jnp.sum(push_sum)) / jnp.maximum(nps, 1.0),
                0.0)
            reg = jnp.full((L,), jnp.sum(regsum)) / kf_safe
            loss = loss + pull + push + REG_W * reg
        ovec[...] = jnp.where(lane == 0, loss, 0.0)
        pltpu.sync_copy(ovec, out_hbm.at[cid])


_sc_kernel = pl.kernel(
    _sc_all,
    out_type=(jax.ShapeDtypeStruct((NCORE, L), jnp.float32),
              jax.ShapeDtypeStruct((NCORE, BPC, NS, K * D), jnp.float32),
              jax.ShapeDtypeStruct((NCORE, BPC, NS, K), jnp.float32),
              jax.ShapeDtypeStruct((NCORE, BPC, NS, K), jnp.float32)),
    mesh=_mesh,
    compiler_params=pltpu.CompilerParams(needs_layout_passes=False),
    scratch_types=[
        pltpu.VMEM((2, D, CHUNK), jnp.float32),
        pltpu.VMEM((2, CHUNK), jnp.int32),
        pltpu.VMEM((K * D,), jnp.float32),
        pltpu.VMEM((K,), jnp.float32),
        pltpu.VMEM((BPC, K), jnp.float32),
        pltpu.VMEM((BPC * K * D,), jnp.float32),
        pltpu.VMEM((K,), jnp.float32),
        pltpu.VMEM((NS, K * D), jnp.float32),
        pltpu.VMEM((NS, K), jnp.float32),
        pltpu.VMEM((L,), jnp.float32),
        pltpu.SemaphoreType.DMA((2,)),
    ],
)


def kernel(pred_embedding, gt_instance, valid_mask):
    del valid_mask
    emb = pred_embedding.reshape(B, D, N)
    gt = gt_instance.reshape(B, N)
    out, _, _, _ = _sc_kernel(emb, gt)
    return jnp.sum(out) / B


# speedup vs baseline: 1.0179x; 1.0179x over previous
import jax
import jax.numpy as jnp
from jax import lax
from jax.experimental import pallas as pl
from jax.experimental.pallas import tpu as pltpu
from jax.experimental.pallas import tpu_sc as plsc

B, D, H, W = 4, 16, 512, 512
K = 32
N = H * W
NCORE, NS, L = 2, 16, 16
BPC = B // NCORE
SPAN = N // NS
CHUNK = 2048
NCK = SPAN // CHUNK
NPAIR = NCK // 2
GRP = CHUNK // L

DELTA_VAR = 0.5
DELTA_DIST = 1.5
REG_W = 0.001

_mesh = plsc.VectorSubcoreMesh(core_axis_name="c", subcore_axis_name="s")


def _sqrt16(s2):
    s2 = jnp.maximum(s2, 1e-30)
    y = plsc.bitcast(0x5F3759DF - (plsc.bitcast(s2, jnp.int32) >> 1),
                     jnp.float32)
    for _ in range(2):
        y = y * (1.5 - 0.5 * s2 * y * y)
    return s2 * y


def _sc_all(emb_hbm, gt_hbm, out_hbm, st_sums, st_cnt, st_pacc, ebuf,
            gbuf, sums, cnt, cnt2, cbuf, pacc, tmp, tmpc, ovec, sems):
    cid = lax.axis_index("c")
    sid = lax.axis_index("s")
    zeros = jnp.zeros((L,), jnp.float32)
    ones = jnp.ones((L,), jnp.float32)
    lane = lax.iota(jnp.int32, L)

    def issue(bb, ck, buf):
        bg = cid * BPC + bb
        off = sid * SPAN + ck * CHUNK
        de = pltpu.async_copy(emb_hbm.at[bg, :, pl.ds(off, CHUNK)],
                              ebuf.at[buf], sems.at[buf])
        dg = pltpu.async_copy(gt_hbm.at[bg, pl.ds(off, CHUNK)],
                              gbuf.at[buf], sems.at[buf])
        return de, dg

    def wait(bb, ck, buf):
        bg = cid * BPC + bb
        off = sid * SPAN + ck * CHUNK
        pltpu.make_async_copy(emb_hbm.at[bg, :, pl.ds(off, CHUNK)],
                              ebuf.at[buf], sems.at[buf]).wait()
        pltpu.make_async_copy(gt_hbm.at[bg, pl.ds(off, CHUNK)],
                              gbuf.at[buf], sems.at[buf]).wait()

    def stream(bb, compute):
        issue(bb, 0, 0)
        issue(bb, 1, 1)

        def pair(p, _):
            wait(bb, 2 * p, 0)
            compute(0)
            issue(bb, 2 * p + 2, 0)
            wait(bb, 2 * p + 1, 1)
            compute(1)
            issue(bb, 2 * p + 3, 1)
            return 0

        lax.fori_loop(0, NPAIR - 1, pair, 0)
        wait(bb, NCK - 2, 0)
        compute(0)
        wait(bb, NCK - 1, 1)
        compute(1)

    for bb in range(BPC):
        for j in range(K * D // L):
            sums[pl.ds(j * L, L)] = zeros
        for j in range(K // L):
            cnt[pl.ds(j * L, L)] = zeros

        def compute1(buf):
            @plsc.parallel_loop(0, GRP, unroll=3)
            def _(i):
                o = i * L
                seg = gbuf[buf, pl.ds(o, L)]
                segd = seg << 4
                vs = [ebuf[buf, d, pl.ds(o, L)] for d in range(D)]
                plsc.addupdate_scatter(cnt, [seg], ones)
                for d in range(D):
                    plsc.addupdate_scatter(sums, [segd + d], vs[d])

        stream(bb, compute1)
        pltpu.sync_copy(sums, st_sums.at[cid, bb, sid])
        pltpu.sync_copy(cnt, st_cnt.at[cid, bb, sid])

    plsc.subcore_barrier()

    for bb in range(BPC):
        pltpu.sync_copy(st_sums.at[cid, bb], tmp)
        pltpu.sync_copy(st_cnt.at[cid, bb], tmpc)

        def red_sums(j, _):
            acc = tmp[0, pl.ds(j * L, L)]
            for w in range(1, NS):
                acc = acc + tmp[w, pl.ds(j * L, L)]
            sums[pl.ds(j * L, L)] = acc
            return 0

        lax.fori_loop(0, K * D // L, red_sums, 0)
        for j in range(K // L):
            acc = tmpc[0, pl.ds(j * L, L)]
            for w in range(1, NS):
                acc = acc + tmpc[w, pl.ds(j * L, L)]
            cnt2[bb, pl.ds(j * L, L)] = acc

        def cen(k, _, bb=bb):
            ck = plsc.load_gather(cnt2, [jnp.full((L,), bb, jnp.int32),
                                         jnp.full((L,), k, jnp.int32)])
            safe = jnp.where(ck > 0.0, ck, 1.0)
            cbuf[pl.ds(bb * K * D + k * D, L)] = (
                sums[pl.ds(k * D, L)] / safe)
            return 0

        lax.fori_loop(0, K, cen, 0)

    for bb in range(BPC):
        for j in range(K // L):
            pacc[pl.ds(j * L, L)] = zeros

        def compute2(buf, bb=bb):
            @plsc.parallel_loop(0, GRP, unroll=3)
            def _(i):
                o = i * L
                seg = gbuf[buf, pl.ds(o, L)]
                segd = seg << 4
                vs = [ebuf[buf, d, pl.ds(o, L)] for d in range(D)]
                cs = [plsc.load_gather(cbuf, [segd + (bb * K * D + d)])
                      for d in range(D)]
                ps = []
                for j in range(4):
                    t0 = vs[j] - cs[j]
                    p = t0 * t0
                    for d in range(j + 4, D, 4):
                        td = vs[d] - cs[d]
                        p = p + td * td
                    ps.append(p)
                s2 = (ps[0] + ps[1]) + (ps[2] + ps[3])
                r = jnp.maximum(_sqrt16(s2) - DELTA_VAR, 0.0)
                plsc.addupdate_scatter(pacc, [seg], r * r)

        stream(bb, compute2)
        pltpu.sync_copy(pacc, st_pacc.at[cid, bb, sid])

    plsc.subcore_barrier()

    @pl.when(sid == 0)
    def _final():
        loss = zeros
        for bb in range(BPC):
            pltpu.sync_copy(st_pacc.at[cid, bb], tmpc)
            kf = zeros
            pullnum = zeros
            pres = []
            for j in range(K // L):
                pinst = tmpc[0, pl.ds(j * L, L)]
                for w in range(1, NS):
                    pinst = pinst + tmpc[w, pl.ds(j * L, L)]
                ck = cnt2[bb, pl.ds(j * L, L)]
                p = (ck > 0.0).astype(jnp.float32)
                pres.append(p)
                safe = jnp.where(ck > 0.0, ck, 1.0)
                kf = kf + p
                pullnum = pullnum + pinst / safe
            kf_safe = jnp.maximum(jnp.full((L,), jnp.sum(kf)), 1.0)
            pull = jnp.full((L,), jnp.sum(pullnum)) / kf_safe

            cj = [[plsc.load_gather(
                cbuf, [(jb * L + lane) * D + (bb * K * D + d)])
                for jb in range(K // L)] for d in range(D)]
            push_sum = zeros
            npair = zeros
            regsum = zeros
            for jb in range(K // L):
                n2 = cj[0][jb] * cj[0][jb]
                for d in range(1, D):
                    n2 = n2 + cj[d][jb] * cj[d][jb]
                regsum = regsum + pres[jb] * _sqrt16(n2)

            def pair_body(i, carry, bb=bb, pres=pres, cj=cj):
                push_sum, npair = carry
                sp = [plsc.load_gather(
                    cbuf, [jnp.full((L,), bb * K * D, jnp.int32)
                           + i * D + d]) for d in range(D)]
                pi = (plsc.load_gather(
                    cnt2, [jnp.full((L,), bb, jnp.int32),
                           jnp.full((L,), 0, jnp.int32) + i])
                    > 0.0).astype(jnp.float32)
                for jb in range(K // L):
                    t0 = sp[0] - cj[0][jb]
                    d2 = t0 * t0
                    for d in range(1, D):
                        td = sp[d] - cj[d][jb]
                        d2 = d2 + td * td
                    dist = _sqrt16(d2)
                    notself = (lane + (jb * L) != i).astype(jnp.float32)
                    pm = pres[jb] * pi * notself
                    rr = jnp.maximum(2.0 * DELTA_DIST - dist, 0.0)
                    push_sum = push_sum + pm * (rr * rr)
                    npair = npair + pm
                return push_sum, npair

            push_sum, npair = lax.fori_loop(0, K, pair_body,
                                            (push_sum, npair))
            nps = jnp.full((L,), jnp.sum(npair))
            push = jnp.where(
                nps > 0.0,
                jnp.full((L,), jnp.sum(push_sum)) / jnp.maximum(nps, 1.0),
                0.0)
            reg = jnp.full((L,), jnp.sum(regsum)) / kf_safe
            loss = loss + pull + push + REG_W * reg
        ovec[...] = jnp.where(lane == 0, loss, 0.0)
        pltpu.sync_copy(ovec, out_hbm.at[cid])


_sc_kernel = pl.kernel(
    _sc_all,
    out_type=(jax.ShapeDtypeStruct((NCORE, L), jnp.float32),
              jax.ShapeDtypeStruct((NCORE, BPC, NS, K * D), jnp.float32),
              jax.ShapeDtypeStruct((NCORE, BPC, NS, K), jnp.float32),
              jax.ShapeDtypeStruct((NCORE, BPC, NS, K), jnp.float32)),
    mesh=_mesh,
    compiler_params=pltpu.CompilerParams(needs_layout_passes=False),
    scratch_types=[
        pltpu.VMEM((2, D, CHUNK), jnp.float32),
        pltpu.VMEM((2, CHUNK), jnp.int32),
        pltpu.VMEM((K * D,), jnp.float32),
        pltpu.VMEM((K,), jnp.float32),
        pltpu.VMEM((BPC, K), jnp.float32),
        pltpu.VMEM((BPC * K * D,), jnp.float32),
        pltpu.VMEM((K,), jnp.float32),
        pltpu.VMEM((NS, K * D), jnp.float32),
        pltpu.VMEM((NS, K), jnp.float32),
        pltpu.VMEM((L,), jnp.float32),
        pltpu.SemaphoreType.DMA((2,)),
    ],
)


def kernel(pred_embedding, gt_instance, valid_mask):
    del valid_mask
    emb = pred_embedding.reshape(B, D, N)
    gt = gt_instance.reshape(B, N)
    out, _, _, _ = _sc_kernel(emb, gt)
    return jnp.sum(out) / B


# speedup vs baseline: 1.0262x; 1.0081x over previous
import jax
import jax.numpy as jnp
from jax import lax
from jax.experimental import pallas as pl
from jax.experimental.pallas import tpu as pltpu
from jax.experimental.pallas import tpu_sc as plsc

B, D, H, W = 4, 16, 512, 512
K = 32
N = H * W
NCORE, NS, L = 2, 16, 16
BPC = B // NCORE
SPAN = N // NS
CHUNK = 2048
NCK = SPAN // CHUNK
NPAIR = NCK // 2
GRP = CHUNK // L

DELTA_VAR = 0.5
DELTA_DIST = 1.5
REG_W = 0.001

_mesh = plsc.VectorSubcoreMesh(core_axis_name="c", subcore_axis_name="s")


def _sqrt16(s2):
    s2 = jnp.maximum(s2, 1e-30)
    y = plsc.bitcast(0x5F3759DF - (plsc.bitcast(s2, jnp.int32) >> 1),
                     jnp.float32)
    for _ in range(2):
        y = y * (1.5 - 0.5 * s2 * y * y)
    return s2 * y


def _sc_all(emb_hbm, gt_hbm, out_hbm, st_sums, st_cnt, st_pacc, ebuf,
            gbuf, sums, cnt, cnt2, cbuf, pacc, tmp, tmpc, ovec, sems):
    cid = lax.axis_index("c")
    sid = lax.axis_index("s")
    zeros = jnp.zeros((L,), jnp.float32)
    ones = jnp.ones((L,), jnp.float32)
    lane = lax.iota(jnp.int32, L)

    def issue(bb, ck, buf):
        bg = cid * BPC + bb
        off = sid * SPAN + ck * CHUNK
        de = pltpu.async_copy(emb_hbm.at[bg, :, pl.ds(off, CHUNK)],
                              ebuf.at[buf], sems.at[buf])
        dg = pltpu.async_copy(gt_hbm.at[bg, pl.ds(off, CHUNK)],
                              gbuf.at[buf], sems.at[buf])
        return de, dg

    def wait(bb, ck, buf):
        bg = cid * BPC + bb
        off = sid * SPAN + ck * CHUNK
        pltpu.make_async_copy(emb_hbm.at[bg, :, pl.ds(off, CHUNK)],
                              ebuf.at[buf], sems.at[buf]).wait()
        pltpu.make_async_copy(gt_hbm.at[bg, pl.ds(off, CHUNK)],
                              gbuf.at[buf], sems.at[buf]).wait()

    def stream(bb, compute):
        issue(bb, 0, 0)
        issue(bb, 1, 1)

        def pair(p, _):
            wait(bb, 2 * p, 0)
            compute(0)
            issue(bb, 2 * p + 2, 0)
            wait(bb, 2 * p + 1, 1)
            compute(1)
            issue(bb, 2 * p + 3, 1)
            return 0

        lax.fori_loop(0, NPAIR - 1, pair, 0)
        wait(bb, NCK - 2, 0)
        compute(0)
        wait(bb, NCK - 1, 1)
        compute(1)

    for bb in range(BPC):
        for j in range(K * D // L):
            sums[pl.ds(j * L, L)] = zeros
        for j in range(K // L):
            cnt[pl.ds(j * L, L)] = zeros

        def compute1(buf):
            @plsc.parallel_loop(0, GRP, unroll=2)
            def _(i):
                o = i * L
                seg = gbuf[buf, pl.ds(o, L)]
                segd = seg << 4
                vs = [ebuf[buf, d, pl.ds(o, L)] for d in range(D)]
                plsc.addupdate_scatter(cnt, [seg], ones)
                for d in range(D):
                    plsc.addupdate_scatter(sums, [segd + d], vs[d])

        stream(bb, compute1)
        pltpu.sync_copy(sums, st_sums.at[cid, bb, sid])
        pltpu.sync_copy(cnt, st_cnt.at[cid, bb, sid])

    plsc.subcore_barrier()

    for bb in range(BPC):
        pltpu.sync_copy(st_sums.at[cid, bb], tmp)
        pltpu.sync_copy(st_cnt.at[cid, bb], tmpc)

        def red_sums(j, _):
            acc = tmp[0, pl.ds(j * L, L)]
            for w in range(1, NS):
                acc = acc + tmp[w, pl.ds(j * L, L)]
            sums[pl.ds(j * L, L)] = acc
            return 0

        lax.fori_loop(0, K * D // L, red_sums, 0)
        for j in range(K // L):
            acc = tmpc[0, pl.ds(j * L, L)]
            for w in range(1, NS):
                acc = acc + tmpc[w, pl.ds(j * L, L)]
            cnt2[bb, pl.ds(j * L, L)] = acc

        def cen(k, _, bb=bb):
            ck = plsc.load_gather(cnt2, [jnp.full((L,), bb, jnp.int32),
                                         jnp.full((L,), k, jnp.int32)])
            safe = jnp.where(ck > 0.0, ck, 1.0)
            cbuf[pl.ds(bb * K * D + k * D, L)] = (
                sums[pl.ds(k * D, L)] / safe)
            return 0

        lax.fori_loop(0, K, cen, 0)

    for bb in range(BPC):
        for j in range(K // L):
            pacc[pl.ds(j * L, L)] = zeros

        def compute2(buf, bb=bb):
            @plsc.parallel_loop(0, GRP, unroll=2)
            def _(i):
                o = i * L
                seg = gbuf[buf, pl.ds(o, L)]
                segd = seg << 4
                vs = [ebuf[buf, d, pl.ds(o, L)] for d in range(D)]
                cs = [plsc.load_gather(cbuf, [segd + (bb * K * D + d)])
                      for d in range(D)]
                ps = []
                for j in range(4):
                    t0 = vs[j] - cs[j]
                    p = t0 * t0
                    for d in range(j + 4, D, 4):
                        td = vs[d] - cs[d]
                        p = p + td * td
                    ps.append(p)
                s2 = (ps[0] + ps[1]) + (ps[2] + ps[3])
                r = jnp.maximum(_sqrt16(s2) - DELTA_VAR, 0.0)
                plsc.addupdate_scatter(pacc, [seg], r * r)

        stream(bb, compute2)
        pltpu.sync_copy(pacc, st_pacc.at[cid, bb, sid])

    plsc.subcore_barrier()

    @pl.when(sid == 0)
    def _final():
        loss = zeros
        for bb in range(BPC):
            pltpu.sync_copy(st_pacc.at[cid, bb], tmpc)
            kf = zeros
            pullnum = zeros
            pres = []
            for j in range(K // L):
                pinst = tmpc[0, pl.ds(j * L, L)]
                for w in range(1, NS):
                    pinst = pinst + tmpc[w, pl.ds(j * L, L)]
                ck = cnt2[bb, pl.ds(j * L, L)]
                p = (ck > 0.0).astype(jnp.float32)
                pres.append(p)
                safe = jnp.where(ck > 0.0, ck, 1.0)
                kf = kf + p
                pullnum = pullnum + pinst / safe
            kf_safe = jnp.maximum(jnp.full((L,), jnp.sum(kf)), 1.0)
            pull = jnp.full((L,), jnp.sum(pullnum)) / kf_safe

            cj = [[plsc.load_gather(
                cbuf, [(jb * L + lane) * D + (bb * K * D + d)])
                for jb in range(K // L)] for d in range(D)]
            push_sum = zeros
            npair = zeros
            regsum = zeros
            for jb in range(K // L):
                n2 = cj[0][jb] * cj[0][jb]
                for d in range(1, D):
                    n2 = n2 + cj[d][jb] * cj[d][jb]
                regsum = regsum + pres[jb] * _sqrt16(n2)

            def pair_body(i, carry, bb=bb, pres=pres, cj=cj):
                push_sum, npair = carry
                sp = [plsc.load_gather(
                    cbuf, [jnp.full((L,), bb * K * D, jnp.int32)
                           + i * D + d]) for d in range(D)]
                pi = (plsc.load_gather(
                    cnt2, [jnp.full((L,), bb, jnp.int32),
                           jnp.full((L,), 0, jnp.int32) + i])
                    > 0.0).astype(jnp.float32)
                for jb in range(K // L):
                    t0 = sp[0] - cj[0][jb]
                    d2 = t0 * t0
                    for d in range(1, D):
                        td = sp[d] - cj[d][jb]
                        d2 = d2 + td * td
                    dist = _sqrt16(d2)
                    notself = (lane + (jb * L) != i).astype(jnp.float32)
                    pm = pres[jb] * pi * notself
                    rr = jnp.maximum(2.0 * DELTA_DIST - dist, 0.0)
                    push_sum = push_sum + pm * (rr * rr)
                    npair = npair + pm
                return push_sum, npair

            push_sum, npair = lax.fori_loop(0, K, pair_body,
                                            (push_sum, npair))
            nps = jnp.full((L,), jnp.sum(npair))
            push = jnp.where(
                nps > 0.0,
                jnp.full((L,), jnp.sum(push_sum)) / jnp.maximum(nps, 1.0),
                0.0)
            reg = jnp.full((L,), jnp.sum(regsum)) / kf_safe
            loss = loss + pull + push + REG_W * reg
        ovec[...] = jnp.where(lane == 0, loss, 0.0)
        pltpu.sync_copy(ovec, out_hbm.at[cid])


_sc_kernel = pl.kernel(
    _sc_all,
    out_type=(jax.ShapeDtypeStruct((NCORE, L), jnp.float32),
              jax.ShapeDtypeStruct((NCORE, BPC, NS, K * D), jnp.float32),
              jax.ShapeDtypeStruct((NCORE, BPC, NS, K), jnp.float32),
              jax.ShapeDtypeStruct((NCORE, BPC, NS, K), jnp.float32)),
    mesh=_mesh,
    compiler_params=pltpu.CompilerParams(needs_layout_passes=False),
    scratch_types=[
        pltpu.VMEM((2, D, CHUNK), jnp.float32),
        pltpu.VMEM((2, CHUNK), jnp.int32),
        pltpu.VMEM((K * D,), jnp.float32),
        pltpu.VMEM((K,), jnp.float32),
        pltpu.VMEM((BPC, K), jnp.float32),
        pltpu.VMEM((BPC * K * D,), jnp.float32),
        pltpu.VMEM((K,), jnp.float32),
        pltpu.VMEM((NS, K * D), jnp.float32),
        pltpu.VMEM((NS, K), jnp.float32),
        pltpu.VMEM((L,), jnp.float32),
        pltpu.SemaphoreType.DMA((2,)),
    ],
)


def kernel(pred_embedding, gt_instance, valid_mask):
    del valid_mask
    emb = pred_embedding.reshape(B, D, N)
    gt = gt_instance.reshape(B, N)
    out, _, _, _ = _sc_kernel(emb, gt)
    return jnp.sum(out) / B
